# baseline (speedup 1.0000x reference)
import functools

import jax
import jax.numpy as jnp
from jax import lax
from jax.experimental import pallas as pl
from jax.experimental.pallas import tpu as pltpu
from jax.experimental.pallas import tpu_sc as plsc

VOCAB = 1000000
EMB = 64
HID = 256
OUT = 100
B = 16384
L = 50

NUM_CORES = 2
NUM_SUBCORES = 16
NW = NUM_CORES * NUM_SUBCORES
BPW = B // NW
NBUF = 8
NVREG = EMB // 16


def _pool_body(idx_hbm, table_hbm, out_hbm, idx_v, rows_v, pooled_v, *sems):
    wid = lax.axis_index("s") * NUM_CORES + lax.axis_index("c")
    base = wid * BPW

    pltpu.sync_copy(idx_hbm.at[pl.ds(base, BPW)], idx_v)

    for b in range(NBUF):
        pltpu.async_copy(table_hbm.at[idx_v.at[b]], rows_v.at[b], sems[b])

    inv_l = jnp.float32(1.0 / L)

    def outer(j, carry):
        for b in range(NBUF):
            i = j * NBUF + b
            buf = rows_v.at[b]
            sem = sems[b]
            pltpu.make_async_copy(table_hbm.at[idx_v.at[i]], buf, sem).wait()
            accs = [buf[0, pl.ds(16 * k, 16)] for k in range(NVREG)]
            for r in range(1, L):
                for k in range(NVREG):
                    accs[k] = accs[k] + buf[r, pl.ds(16 * k, 16)]
            for k in range(NVREG):
                pooled_v[i, pl.ds(16 * k, 16)] = accs[k] * inv_l
            @pl.when(j < BPW // NBUF - 1)
            def _():
                pltpu.async_copy(table_hbm.at[idx_v.at[i + NBUF]], buf, sem)
        return carry

    lax.fori_loop(0, BPW // NBUF, outer, 0, unroll=False)

    pltpu.sync_copy(pooled_v, out_hbm.at[pl.ds(base, BPW)])


@jax.jit
def _gather_pool(idx, emb_table):
    mesh = plsc.VectorSubcoreMesh(core_axis_name="c", subcore_axis_name="s")
    return pl.kernel(
        _pool_body,
        out_type=jax.ShapeDtypeStruct((B, EMB), jnp.float32),
        mesh=mesh,
        scratch_types=[
            pltpu.VMEM((BPW, L), jnp.int32),
            pltpu.VMEM((NBUF, L, EMB), jnp.float32),
            pltpu.VMEM((BPW, EMB), jnp.float32),
        ] + [pltpu.SemaphoreType.DMA] * NBUF,
        compiler_params=pltpu.CompilerParams(use_tc_tiling_on_sc=False),
    )(idx, emb_table)


def _mlp_body(x_ref, w1_ref, b1_ref, w2_ref, b2_ref, o_ref):
    x = x_ref[...]
    h = jnp.dot(x, w1_ref[...], preferred_element_type=jnp.float32)
    h = jnp.maximum(h + b1_ref[...], 0.0)
    logits = jnp.dot(h, w2_ref[...], preferred_element_type=jnp.float32)
    logits = logits + b2_ref[...]
    m = jnp.max(logits, axis=-1, keepdims=True)
    e = jnp.exp(logits - m)
    lse = jnp.log(jnp.sum(e, axis=-1, keepdims=True)) + m
    o_ref[...] = logits - lse


def _mlp(pooled, W1, b1, W2, b2):
    BM = 2048
    grid = (B // BM,)
    return pl.pallas_call(
        _mlp_body,
        grid=grid,
        in_specs=[
            pl.BlockSpec((BM, EMB), lambda i: (i, 0)),
            pl.BlockSpec((EMB, HID), lambda i: (0, 0)),
            pl.BlockSpec((1, HID), lambda i: (0, 0)),
            pl.BlockSpec((HID, OUT), lambda i: (0, 0)),
            pl.BlockSpec((1, OUT), lambda i: (0, 0)),
        ],
        out_specs=pl.BlockSpec((BM, OUT), lambda i: (i, 0)),
        out_shape=jax.ShapeDtypeStruct((B, OUT), jnp.float32),
    )(pooled, W1, b1.reshape(1, HID), W2, b2.reshape(1, OUT))


def kernel(input, emb_table, W1, b1, W2, b2):
    pooled = _gather_pool(input.astype(jnp.int32), emb_table)
    return _mlp(pooled, W1, b1, W2, b2)


# speedup vs baseline: 1.0889x; 1.0889x over previous
import jax
import jax.numpy as jnp
from jax import lax
from jax.experimental import pallas as pl
from jax.experimental.pallas import tpu as pltpu
from jax.experimental.pallas import tpu_sc as plsc

VOCAB = 1000000
EMB = 64
HID = 256
OUT = 100
B = 16384
L = 50
PADDED = 128

NUM_CORES = 2
NUM_SUBCORES = 16
NW = NUM_CORES * NUM_SUBCORES
BPW = B // NW
NBUF = 8
NVREG = EMB // 16


def _pool_body(idx_hbm, table_hbm, out_hbm, idx_v, rows_v, pooled_v, *sems):
    wid = lax.axis_index("s") * NUM_CORES + lax.axis_index("c")
    base = wid * BPW

    pltpu.sync_copy(idx_hbm.at[pl.ds(base, BPW)], idx_v)

    for b in range(NBUF):
        pltpu.async_copy(table_hbm.at[idx_v.at[b]], rows_v.at[b], sems[b])

    inv_l = jnp.float32(1.0 / L)

    def outer(j, carry):
        for bb in range(NBUF):
            i = j * NBUF + bb
            buf = rows_v.at[bb]
            pltpu.make_async_copy(
                table_hbm.at[idx_v.at[i]], buf, sems[bb]).wait()
            accs = [buf[0, pl.ds(16 * k, 16)] for k in range(NVREG)]
            for r in range(1, L):
                for k in range(NVREG):
                    accs[k] = accs[k] + buf[r, pl.ds(16 * k, 16)]
            for k in range(NVREG):
                pooled_v[bb, pl.ds(16 * k, 16)] = accs[k] * inv_l
            @pl.when(i + NBUF < BPW)
            def _(i=i, bb=bb):
                pltpu.async_copy(
                    table_hbm.at[idx_v.at[i + NBUF]], buf, sems[bb])
        pltpu.sync_copy(pooled_v, out_hbm.at[pl.ds(base + j * NBUF, NBUF)])
        return carry

    lax.fori_loop(0, BPW // NBUF, outer, 0, unroll=False)


@jax.jit
def _gather_pool(idx, table_pad):
    mesh = plsc.VectorSubcoreMesh(core_axis_name="c", subcore_axis_name="s")
    return pl.kernel(
        _pool_body,
        out_type=jax.ShapeDtypeStruct((B, EMB), jnp.float32),
        mesh=mesh,
        scratch_types=[
            pltpu.VMEM((BPW, L), jnp.int32),
            pltpu.VMEM((NBUF, L, PADDED), jnp.float32),
            pltpu.VMEM((NBUF, EMB), jnp.float32),
        ] + [pltpu.SemaphoreType.DMA] * NBUF,
        compiler_params=pltpu.CompilerParams(use_tc_tiling_on_sc=True),
    )(idx, table_pad)


def _mlp_body(x_ref, w1_ref, b1_ref, w2_ref, b2_ref, o_ref):
    x = x_ref[...]
    h = jnp.dot(x, w1_ref[...], preferred_element_type=jnp.float32)
    h = jnp.maximum(h + b1_ref[...], 0.0)
    logits = jnp.dot(h, w2_ref[...], preferred_element_type=jnp.float32)
    logits = logits + b2_ref[...]
    m = jnp.max(logits, axis=-1, keepdims=True)
    e = jnp.exp(logits - m)
    lse = jnp.log(jnp.sum(e, axis=-1, keepdims=True)) + m
    o_ref[...] = logits - lse


def _mlp(pooled, W1, b1, W2, b2):
    BM = 2048
    grid = (B // BM,)
    return pl.pallas_call(
        _mlp_body,
        grid=grid,
        in_specs=[
            pl.BlockSpec((BM, EMB), lambda i: (i, 0)),
            pl.BlockSpec((EMB, HID), lambda i: (0, 0)),
            pl.BlockSpec((1, HID), lambda i: (0, 0)),
            pl.BlockSpec((HID, OUT), lambda i: (0, 0)),
            pl.BlockSpec((1, OUT), lambda i: (0, 0)),
        ],
        out_specs=pl.BlockSpec((BM, OUT), lambda i: (i, 0)),
        out_shape=jax.ShapeDtypeStruct((B, OUT), jnp.float32),
    )(pooled, W1, b1.reshape(1, HID), W2, b2.reshape(1, OUT))


def kernel(input, emb_table, W1, b1, W2, b2):
    table_pad = jnp.pad(emb_table, ((0, 0), (0, PADDED - EMB)))
    pooled = _gather_pool(input.astype(jnp.int32), table_pad)
    return _mlp(pooled, W1, b1, W2, b2)


# speedup vs baseline: 1.1953x; 1.0977x over previous
import jax
import jax.numpy as jnp
from jax import lax
from jax.experimental import pallas as pl
from jax.experimental.pallas import tpu as pltpu
from jax.experimental.pallas import tpu_sc as plsc

VOCAB = 1000000
EMB = 64
HID = 256
OUT = 100
B = 16384
L = 50
PADDED = 128

NUM_CORES = 2
NUM_SUBCORES = 16
NW = NUM_CORES * NUM_SUBCORES
BPW = B // NW
NBUF = 8
NVREG = EMB // 16


def _pool_body(idx_hbm, table_hbm, out_hbm, idx_v, rows_v, pooled_v, *sems):
    wid = lax.axis_index("s") * NUM_CORES + lax.axis_index("c")
    base = wid * BPW

    pltpu.sync_copy(idx_hbm.at[pl.ds(base, BPW)], idx_v)

    for b in range(NBUF):
        pltpu.async_copy(table_hbm.at[idx_v.at[b]], rows_v.at[b], sems[b])

    inv_l = jnp.float32(1.0 / L)

    def outer(j, carry):
        for bb in range(NBUF):
            i = j * NBUF + bb
            buf = rows_v.at[bb]
            pltpu.make_async_copy(
                table_hbm.at[idx_v.at[i]], buf, sems[bb]).wait()
            accs = [buf[0, pl.ds(16 * k, 16)] for k in range(NVREG)]
            for r in range(1, L):
                for k in range(NVREG):
                    accs[k] = accs[k] + buf[r, pl.ds(16 * k, 16)]
            for k in range(NVREG):
                pooled_v[bb, pl.ds(16 * k, 16)] = accs[k] * inv_l
            @pl.when(i + NBUF < BPW)
            def _(i=i, bb=bb):
                pltpu.async_copy(
                    table_hbm.at[idx_v.at[i + NBUF]], buf, sems[bb])
        pltpu.sync_copy(pooled_v, out_hbm.at[pl.ds(base + j * NBUF, NBUF)])
        return carry

    lax.fori_loop(0, BPW // NBUF, outer, 0, unroll=False)


@jax.jit
def _gather_pool(idx, table_pad):
    mesh = plsc.VectorSubcoreMesh(core_axis_name="c", subcore_axis_name="s")
    return pl.kernel(
        _pool_body,
        out_type=jax.ShapeDtypeStruct((B, EMB), jnp.float32),
        mesh=mesh,
        scratch_types=[
            pltpu.VMEM((BPW, L), jnp.int32),
            pltpu.VMEM((NBUF, L, PADDED), jnp.float32),
            pltpu.VMEM((NBUF, EMB), jnp.float32),
        ] + [pltpu.SemaphoreType.DMA] * NBUF,
        compiler_params=pltpu.CompilerParams(use_tc_tiling_on_sc=True),
    )(idx, table_pad)


def _pad_body(t_ref, o_ref):
    x = t_ref[...]
    y = jnp.transpose(x, (1, 0))
    o_ref[...] = jnp.concatenate(
        [y, jnp.zeros_like(y)], axis=1)


_PAD_CW = 2048


def _pad_table(emb_table_t):
    grid = ((VOCAB + _PAD_CW - 1) // _PAD_CW,)
    return pl.pallas_call(
        _pad_body,
        grid=grid,
        in_specs=[pl.BlockSpec((EMB, _PAD_CW), lambda i: (0, i))],
        out_specs=pl.BlockSpec((_PAD_CW, PADDED), lambda i: (i, 0)),
        out_shape=jax.ShapeDtypeStruct((VOCAB, PADDED), jnp.float32),
    )(emb_table_t)


def _mlp_body(x_ref, w1_ref, b1_ref, w2_ref, b2_ref, o_ref):
    x = x_ref[...]
    h = jnp.dot(x, w1_ref[...], preferred_element_type=jnp.float32)
    h = jnp.maximum(h + b1_ref[...], 0.0)
    logits = jnp.dot(h, w2_ref[...], preferred_element_type=jnp.float32)
    logits = logits + b2_ref[...]
    m = jnp.max(logits, axis=-1, keepdims=True)
    e = jnp.exp(logits - m)
    lse = jnp.log(jnp.sum(e, axis=-1, keepdims=True)) + m
    o_ref[...] = logits - lse


def _mlp(pooled, W1, b1, W2, b2):
    BM = 2048
    grid = (B // BM,)
    return pl.pallas_call(
        _mlp_body,
        grid=grid,
        in_specs=[
            pl.BlockSpec((BM, EMB), lambda i: (i, 0)),
            pl.BlockSpec((EMB, HID), lambda i: (0, 0)),
            pl.BlockSpec((1, HID), lambda i: (0, 0)),
            pl.BlockSpec((HID, OUT), lambda i: (0, 0)),
            pl.BlockSpec((1, OUT), lambda i: (0, 0)),
        ],
        out_specs=pl.BlockSpec((BM, OUT), lambda i: (i, 0)),
        out_shape=jax.ShapeDtypeStruct((B, OUT), jnp.float32),
    )(pooled, W1, b1.reshape(1, HID), W2, b2.reshape(1, OUT))


def kernel(input, emb_table, W1, b1, W2, b2):
    table_pad = _pad_table(emb_table.T)
    pooled = _gather_pool(input.astype(jnp.int32), table_pad)
    return _mlp(pooled, W1, b1, W2, b2)


# speedup vs baseline: 1.4864x; 1.2436x over previous
import jax
import jax.numpy as jnp
from jax import lax
from jax.experimental import pallas as pl
from jax.experimental.pallas import tpu as pltpu
from jax.experimental.pallas import tpu_sc as plsc

VOCAB = 1000000
EMB = 64
HID = 256
OUT = 100
B = 16384
L = 50
PADDED = 128

NUM_CORES = 2
NUM_SUBCORES = 16
NW = NUM_CORES * NUM_SUBCORES
EPS = 1
SL = EPS * L
NPAIR = B // EPS
PPW = NPAIR // NW
NBUF = 4
NVREG = EMB // 16


def _pool_body(idx_hbm, table_hbm, out_hbm, idx_v, rows_v, pooled_v, *sems):
    wid = lax.axis_index("s") * NUM_CORES + lax.axis_index("c")
    base = wid * PPW

    pltpu.sync_copy(idx_hbm.at[pl.ds(base, PPW)], idx_v)

    for b in range(NBUF):
        pltpu.async_copy(table_hbm.at[idx_v.at[b]], rows_v.at[b], sems[b])

    inv_l = jnp.float32(1.0 / L)

    def outer(j, carry):
        for bb in range(NBUF):
            i = j * NBUF + bb
            buf = rows_v.at[bb]
            pltpu.make_async_copy(
                table_hbm.at[idx_v.at[i]], buf, sems[bb]).wait()
            for s in range(EPS):
                accs = [buf[s * L, pl.ds(16 * k, 16)] for k in range(NVREG)]
                for r in range(1, L):
                    for k in range(NVREG):
                        accs[k] = accs[k] + buf[s * L + r, pl.ds(16 * k, 16)]
                for k in range(NVREG):
                    pooled_v[bb * EPS + s, pl.ds(16 * k, 16)] = accs[k] * inv_l
            @pl.when(i + NBUF < PPW)
            def _(i=i, bb=bb):
                pltpu.async_copy(
                    table_hbm.at[idx_v.at[i + NBUF]], buf, sems[bb])
        pltpu.sync_copy(
            pooled_v,
            out_hbm.at[pl.ds((base + j * NBUF) * EPS, NBUF * EPS)])
        return carry

    lax.fori_loop(0, PPW // NBUF, outer, 0, unroll=False)


@jax.jit
def _gather_pool(idx2, table_pad):
    mesh = plsc.VectorSubcoreMesh(core_axis_name="c", subcore_axis_name="s")
    return pl.kernel(
        _pool_body,
        out_type=jax.ShapeDtypeStruct((B, EMB), jnp.float32),
        mesh=mesh,
        scratch_types=[
            pltpu.VMEM((PPW, SL), jnp.int32),
            pltpu.VMEM((NBUF, SL, PADDED), jnp.float32),
            pltpu.VMEM((NBUF * EPS, EMB), jnp.float32),
        ] + [pltpu.SemaphoreType.DMA] * NBUF,
        compiler_params=pltpu.CompilerParams(use_tc_tiling_on_sc=True),
    )(idx2, table_pad)


def _pad_body(t_ref, o_ref):
    y = jnp.transpose(t_ref[...], (1, 0))
    o_ref[...] = jnp.concatenate([y, jnp.zeros_like(y)], axis=1)


_PAD_CW = 4096


def _pad_table(emb_table_t):
    grid = ((VOCAB + _PAD_CW - 1) // _PAD_CW,)
    return pl.pallas_call(
        _pad_body,
        grid=grid,
        in_specs=[pl.BlockSpec((EMB, _PAD_CW), lambda i: (0, i))],
        out_specs=pl.BlockSpec((_PAD_CW, PADDED), lambda i: (i, 0)),
        out_shape=jax.ShapeDtypeStruct((VOCAB, PADDED), jnp.float32),
    )(emb_table_t)


def _mlp_body(x_ref, w1_ref, b1_ref, w2_ref, b2_ref, o_ref):
    x = x_ref[...]
    h = jnp.dot(x, w1_ref[...], preferred_element_type=jnp.float32)
    h = jnp.maximum(h + b1_ref[...], 0.0)
    logits = jnp.dot(h, w2_ref[...], preferred_element_type=jnp.float32)
    logits = logits + b2_ref[...]
    m = jnp.max(logits, axis=-1, keepdims=True)
    e = jnp.exp(logits - m)
    lse = jnp.log(jnp.sum(e, axis=-1, keepdims=True)) + m
    o_ref[...] = logits - lse


def _mlp(pooled, W1, b1, W2, b2):
    BM = 2048
    grid = (B // BM,)
    return pl.pallas_call(
        _mlp_body,
        grid=grid,
        in_specs=[
            pl.BlockSpec((BM, EMB), lambda i: (i, 0)),
            pl.BlockSpec((EMB, HID), lambda i: (0, 0)),
            pl.BlockSpec((1, HID), lambda i: (0, 0)),
            pl.BlockSpec((HID, OUT), lambda i: (0, 0)),
            pl.BlockSpec((1, OUT), lambda i: (0, 0)),
        ],
        out_specs=pl.BlockSpec((BM, OUT), lambda i: (i, 0)),
        out_shape=jax.ShapeDtypeStruct((B, OUT), jnp.float32),
    )(pooled, W1, b1.reshape(1, HID), W2, b2.reshape(1, OUT))


def kernel(input, emb_table, W1, b1, W2, b2):
    table_pad = _pad_table(emb_table.T)
    pooled = _gather_pool(input.astype(jnp.int32), table_pad)
    return _mlp(pooled, W1, b1, W2, b2)


# speedup vs baseline: 1.7063x; 1.1479x over previous
import jax
import jax.numpy as jnp
from jax import lax
from jax.experimental import pallas as pl
from jax.experimental.pallas import tpu as pltpu
from jax.experimental.pallas import tpu_sc as plsc

VOCAB = 1000000
EMB = 64
HID = 256
OUT = 100
B = 16384
L = 50
PADDED = 128

NUM_CORES = 2
NUM_SUBCORES = 16
NW = NUM_CORES * NUM_SUBCORES
EPS = 1
SL = EPS * L
NPAIR = B // EPS
PPW = NPAIR // NW
NBUF = 4
NVREG = EMB // 16


def _pool_body(idx_hbm, table_hbm, out_hbm, idx_v, rows_v, pooled_v, *sems):
    wid = lax.axis_index("s") * NUM_CORES + lax.axis_index("c")
    base = wid * PPW

    pltpu.sync_copy(idx_hbm.at[pl.ds(base, PPW)], idx_v)

    for b in range(NBUF):
        pltpu.async_copy(table_hbm.at[idx_v.at[b]], rows_v.at[b], sems[b])

    inv_l = jnp.float32(1.0 / L)

    def outer(j, carry):
        for bb in range(NBUF):
            i = j * NBUF + bb
            buf = rows_v.at[bb]
            pltpu.make_async_copy(
                table_hbm.at[idx_v.at[i]], buf, sems[bb]).wait()
            for s in range(EPS):
                accs = [buf[s * L, pl.ds(16 * k, 16)] for k in range(NVREG)]
                for r in range(1, L):
                    for k in range(NVREG):
                        accs[k] = accs[k] + buf[s * L + r, pl.ds(16 * k, 16)]
                for k in range(NVREG):
                    pooled_v[bb * EPS + s, pl.ds(16 * k, 16)] = accs[k] * inv_l
            @pl.when(i + NBUF < PPW)
            def _(i=i, bb=bb):
                pltpu.async_copy(
                    table_hbm.at[idx_v.at[i + NBUF]], buf, sems[bb])
        pltpu.sync_copy(
            pooled_v,
            out_hbm.at[pl.ds((base + j * NBUF) * EPS, NBUF * EPS)])
        return carry

    lax.fori_loop(0, PPW // NBUF, outer, 0, unroll=False)


@jax.jit
def _gather_pool(idx2, table_pad):
    mesh = plsc.VectorSubcoreMesh(core_axis_name="c", subcore_axis_name="s")
    return pl.kernel(
        _pool_body,
        out_type=jax.ShapeDtypeStruct((B, EMB), jnp.float32),
        mesh=mesh,
        scratch_types=[
            pltpu.VMEM((PPW, SL), jnp.int32),
            pltpu.VMEM((NBUF, SL, PADDED), jnp.float32),
            pltpu.VMEM((NBUF * EPS, EMB), jnp.float32),
        ] + [pltpu.SemaphoreType.DMA] * NBUF,
        compiler_params=pltpu.CompilerParams(use_tc_tiling_on_sc=True),
    )(idx2, table_pad)


def _pad_body(t_ref, o_ref):
    y = jnp.transpose(t_ref[...], (1, 0))
    o_ref[...] = jnp.concatenate([y, jnp.zeros_like(y)], axis=1)


_PAD_CW = 8192


def _pad_table(emb_table_t):
    grid = ((VOCAB + _PAD_CW - 1) // _PAD_CW,)
    return pl.pallas_call(
        _pad_body,
        grid=grid,
        in_specs=[pl.BlockSpec((EMB, _PAD_CW), lambda i: (0, i))],
        out_specs=pl.BlockSpec((_PAD_CW, PADDED), lambda i: (i, 0)),
        out_shape=jax.ShapeDtypeStruct((VOCAB, PADDED), jnp.float32),
    )(emb_table_t)


def _mlp_body(x_ref, w1_ref, b1_ref, w2_ref, b2_ref, o_ref):
    x = x_ref[...]
    h = jnp.dot(x, w1_ref[...], preferred_element_type=jnp.float32)
    h = jnp.maximum(h + b1_ref[...], 0.0)
    logits = jnp.dot(h, w2_ref[...], preferred_element_type=jnp.float32)
    logits = logits + b2_ref[...]
    m = jnp.max(logits, axis=-1, keepdims=True)
    e = jnp.exp(logits - m)
    lse = jnp.log(jnp.sum(e, axis=-1, keepdims=True)) + m
    o_ref[...] = logits - lse


def _mlp(pooled, W1, b1, W2, b2):
    BM = 2048
    grid = (B // BM,)
    return pl.pallas_call(
        _mlp_body,
        grid=grid,
        in_specs=[
            pl.BlockSpec((BM, EMB), lambda i: (i, 0)),
            pl.BlockSpec((EMB, HID), lambda i: (0, 0)),
            pl.BlockSpec((1, HID), lambda i: (0, 0)),
            pl.BlockSpec((HID, OUT), lambda i: (0, 0)),
            pl.BlockSpec((1, OUT), lambda i: (0, 0)),
        ],
        out_specs=pl.BlockSpec((BM, OUT), lambda i: (i, 0)),
        out_shape=jax.ShapeDtypeStruct((B, OUT), jnp.float32),
    )(pooled, W1, b1.reshape(1, HID), W2, b2.reshape(1, OUT))


def kernel(input, emb_table, W1, b1, W2, b2):
    table_pad = _pad_table(emb_table.T)
    pooled = _gather_pool(input.astype(jnp.int32), table_pad)
    return _mlp(pooled, W1, b1, W2, b2)


# speedup vs baseline: 1.7768x; 1.0413x over previous
import jax
import jax.numpy as jnp
from jax import lax
from jax.experimental import pallas as pl
from jax.experimental.pallas import tpu as pltpu
from jax.experimental.pallas import tpu_sc as plsc

VOCAB = 1000000
EMB = 64
HID = 256
OUT = 100
B = 16384
L = 50
PADDED = 128

NUM_CORES = 2
NUM_SUBCORES = 16
NW = NUM_CORES * NUM_SUBCORES
EPS = 1
SL = EPS * L
NPAIR = B // EPS
PPW = NPAIR // NW
NBUF = 4
NVREG = EMB // 16


def _pool_body(idx_hbm, table_hbm, out_hbm, idx_v, rows_v, pooled_v, *sems):
    wid = lax.axis_index("s") * NUM_CORES + lax.axis_index("c")
    base = wid * PPW

    pltpu.sync_copy(idx_hbm.at[pl.ds(base, PPW)], idx_v)

    for b in range(NBUF):
        pltpu.async_copy(table_hbm.at[idx_v.at[b]], rows_v.at[b], sems[b])

    inv_l = jnp.float32(1.0 / L)

    def outer(j, carry):
        for bb in range(NBUF):
            i = j * NBUF + bb
            buf = rows_v.at[bb]
            pltpu.make_async_copy(
                table_hbm.at[idx_v.at[i]], buf, sems[bb]).wait()
            for s in range(EPS):
                accs = [buf[s * L, pl.ds(16 * k, 16)] for k in range(NVREG)]
                for r in range(1, L):
                    for k in range(NVREG):
                        accs[k] = accs[k] + buf[s * L + r, pl.ds(16 * k, 16)]
                for k in range(NVREG):
                    pooled_v[bb * EPS + s, pl.ds(16 * k, 16)] = accs[k] * inv_l
            @pl.when(i + NBUF < PPW)
            def _(i=i, bb=bb):
                pltpu.async_copy(
                    table_hbm.at[idx_v.at[i + NBUF]], buf, sems[bb])
        pltpu.sync_copy(
            pooled_v,
            out_hbm.at[pl.ds((base + j * NBUF) * EPS, NBUF * EPS)])
        return carry

    lax.fori_loop(0, PPW // NBUF, outer, 0, unroll=False)


@jax.jit
def _gather_pool(idx2, table_pad):
    mesh = plsc.VectorSubcoreMesh(core_axis_name="c", subcore_axis_name="s")
    return pl.kernel(
        _pool_body,
        out_type=jax.ShapeDtypeStruct((B, EMB), jnp.float32),
        mesh=mesh,
        scratch_types=[
            pltpu.VMEM((PPW, SL), jnp.int32),
            pltpu.VMEM((NBUF, SL, PADDED), jnp.float32),
            pltpu.VMEM((NBUF * EPS, EMB), jnp.float32),
        ] + [pltpu.SemaphoreType.DMA] * NBUF,
        compiler_params=pltpu.CompilerParams(use_tc_tiling_on_sc=True),
    )(idx2, table_pad)


def _pad_body(t_ref, o_ref):
    y = jnp.transpose(t_ref[...], (1, 0))
    o_ref[...] = jnp.concatenate([y, jnp.zeros_like(y)], axis=1)


_PAD_CW = 16384


def _pad_table(emb_table_t):
    grid = ((VOCAB + _PAD_CW - 1) // _PAD_CW,)
    return pl.pallas_call(
        _pad_body,
        grid=grid,
        in_specs=[pl.BlockSpec((EMB, _PAD_CW), lambda i: (0, i))],
        out_specs=pl.BlockSpec((_PAD_CW, PADDED), lambda i: (i, 0)),
        out_shape=jax.ShapeDtypeStruct((VOCAB, PADDED), jnp.float32),
    )(emb_table_t)


def _mlp_body(x_ref, w1_ref, b1_ref, w2_ref, b2_ref, o_ref):
    x = x_ref[...]
    h = jnp.dot(x, w1_ref[...], preferred_element_type=jnp.float32)
    h = jnp.maximum(h + b1_ref[...], 0.0)
    logits = jnp.dot(h, w2_ref[...], preferred_element_type=jnp.float32)
    logits = logits + b2_ref[...]
    m = jnp.max(logits, axis=-1, keepdims=True)
    e = jnp.exp(logits - m)
    lse = jnp.log(jnp.sum(e, axis=-1, keepdims=True)) + m
    o_ref[...] = logits - lse


def _mlp(pooled, W1, b1, W2, b2):
    BM = 2048
    grid = (B // BM,)
    return pl.pallas_call(
        _mlp_body,
        grid=grid,
        in_specs=[
            pl.BlockSpec((BM, EMB), lambda i: (i, 0)),
            pl.BlockSpec((EMB, HID), lambda i: (0, 0)),
            pl.BlockSpec((1, HID), lambda i: (0, 0)),
            pl.BlockSpec((HID, OUT), lambda i: (0, 0)),
            pl.BlockSpec((1, OUT), lambda i: (0, 0)),
        ],
        out_specs=pl.BlockSpec((BM, OUT), lambda i: (i, 0)),
        out_shape=jax.ShapeDtypeStruct((B, OUT), jnp.float32),
    )(pooled, W1, b1.reshape(1, HID), W2, b2.reshape(1, OUT))


def kernel(input, emb_table, W1, b1, W2, b2):
    table_pad = _pad_table(emb_table.T)
    pooled = _gather_pool(input.astype(jnp.int32), table_pad)
    return _mlp(pooled, W1, b1, W2, b2)


# speedup vs baseline: 1.7967x; 1.0112x over previous
import jax
import jax.numpy as jnp
from jax import lax
from jax.experimental import pallas as pl
from jax.experimental.pallas import tpu as pltpu
from jax.experimental.pallas import tpu_sc as plsc

VOCAB = 1000000
EMB = 64
HID = 256
OUT = 100
B = 16384
L = 50
PADDED = 128

NUM_CORES = 2
NUM_SUBCORES = 16
NW = NUM_CORES * NUM_SUBCORES
EPS = 1
SL = EPS * L
NPAIR = B // EPS
PPW = NPAIR // NW
NBUF = 4
NVREG = EMB // 16


def _pool_body(idx_hbm, table_hbm, out_hbm, idx_v, rows_v, pooled_v, *sems):
    wid = lax.axis_index("s") * NUM_CORES + lax.axis_index("c")
    base = wid * PPW

    pltpu.sync_copy(idx_hbm.at[pl.ds(base, PPW)], idx_v)

    for b in range(NBUF):
        pltpu.async_copy(table_hbm.at[idx_v.at[b]], rows_v.at[b], sems[b])

    inv_l = jnp.float32(1.0 / L)

    def outer(j, carry):
        for bb in range(NBUF):
            i = j * NBUF + bb
            buf = rows_v.at[bb]
            pltpu.make_async_copy(
                table_hbm.at[idx_v.at[i]], buf, sems[bb]).wait()
            for s in range(EPS):
                accs = [buf[s * L, pl.ds(16 * k, 16)] for k in range(NVREG)]
                for r in range(1, L):
                    for k in range(NVREG):
                        accs[k] = accs[k] + buf[s * L + r, pl.ds(16 * k, 16)]
                for k in range(NVREG):
                    pooled_v[bb * EPS + s, pl.ds(16 * k, 16)] = accs[k] * inv_l
            @pl.when(i + NBUF < PPW)
            def _(i=i, bb=bb):
                pltpu.async_copy(
                    table_hbm.at[idx_v.at[i + NBUF]], buf, sems[bb])
        pltpu.sync_copy(
            pooled_v,
            out_hbm.at[pl.ds((base + j * NBUF) * EPS, NBUF * EPS)])
        return carry

    lax.fori_loop(0, PPW // NBUF, outer, 0, unroll=False)


@jax.jit
def _gather_pool(idx2, table_pad):
    mesh = plsc.VectorSubcoreMesh(core_axis_name="c", subcore_axis_name="s")
    return pl.kernel(
        _pool_body,
        out_type=jax.ShapeDtypeStruct((B, EMB), jnp.float32),
        mesh=mesh,
        scratch_types=[
            pltpu.VMEM((PPW, SL), jnp.int32),
            pltpu.VMEM((NBUF, SL, PADDED), jnp.float32),
            pltpu.VMEM((NBUF * EPS, EMB), jnp.float32),
        ] + [pltpu.SemaphoreType.DMA] * NBUF,
        compiler_params=pltpu.CompilerParams(use_tc_tiling_on_sc=True),
    )(idx2, table_pad)


def _pad_body(t_ref, o_ref):
    y = jnp.transpose(t_ref[...], (1, 0))
    o_ref[...] = jnp.concatenate([y, jnp.zeros_like(y)], axis=1)


_PAD_CW = 32768


def _pad_table(emb_table_t):
    grid = ((VOCAB + _PAD_CW - 1) // _PAD_CW,)
    return pl.pallas_call(
        _pad_body,
        grid=grid,
        in_specs=[pl.BlockSpec((EMB, _PAD_CW), lambda i: (0, i))],
        out_specs=pl.BlockSpec((_PAD_CW, PADDED), lambda i: (i, 0)),
        out_shape=jax.ShapeDtypeStruct((VOCAB, PADDED), jnp.float32),
    )(emb_table_t)


def _mlp_body(x_ref, w1_ref, b1_ref, w2_ref, b2_ref, o_ref):
    x = x_ref[...]
    h = jnp.dot(x, w1_ref[...], preferred_element_type=jnp.float32)
    h = jnp.maximum(h + b1_ref[...], 0.0)
    logits = jnp.dot(h, w2_ref[...], preferred_element_type=jnp.float32)
    logits = logits + b2_ref[...]
    m = jnp.max(logits, axis=-1, keepdims=True)
    e = jnp.exp(logits - m)
    lse = jnp.log(jnp.sum(e, axis=-1, keepdims=True)) + m
    o_ref[...] = logits - lse


def _mlp(pooled, W1, b1, W2, b2):
    BM = 2048
    grid = (B // BM,)
    return pl.pallas_call(
        _mlp_body,
        grid=grid,
        in_specs=[
            pl.BlockSpec((BM, EMB), lambda i: (i, 0)),
            pl.BlockSpec((EMB, HID), lambda i: (0, 0)),
            pl.BlockSpec((1, HID), lambda i: (0, 0)),
            pl.BlockSpec((HID, OUT), lambda i: (0, 0)),
            pl.BlockSpec((1, OUT), lambda i: (0, 0)),
        ],
        out_specs=pl.BlockSpec((BM, OUT), lambda i: (i, 0)),
        out_shape=jax.ShapeDtypeStruct((B, OUT), jnp.float32),
    )(pooled, W1, b1.reshape(1, HID), W2, b2.reshape(1, OUT))


def kernel(input, emb_table, W1, b1, W2, b2):
    table_pad = _pad_table(emb_table.T)
    pooled = _gather_pool(input.astype(jnp.int32), table_pad)
    return _mlp(pooled, W1, b1, W2, b2)
